# baseline (speedup 1.0000x reference)
import functools
import jax
import jax.numpy as jnp
from jax import lax
from jax.experimental import pallas as pl
from jax.experimental.pallas import tpu as pltpu
from jax.experimental.pallas import tpu_sc as plsc

B, T, D = 4, 2048, 768
E = 8
K = 2
N = B * T
BM = 1024
BG = 2048
NC, NS, L = 2, 16, 16
NW = NC * NS
TPW = N // NW


def _gate_kernel(x_ref, gw_ref, gb_ref, out_ref):
    out_ref[...] = jax.lax.dot_general(
        gw_ref[...], x_ref[...], (((1,), (1,)), ((), ())),
        preferred_element_type=jnp.float32,
    ) + gb_ref[...]


def _expert_kernel(x_ref, ct_ref, ew_ref, eb_ref, out_ref):
    x = x_ref[...]
    combine = ct_ref[...].T
    acc = jax.lax.dot_general(
        combine, eb_ref[...], (((1,), (0,)), ((), ())),
        preferred_element_type=jnp.float32,
    )
    for e in range(E):
        xe = combine[:, e:e + 1] * x
        acc = acc + jax.lax.dot_general(
            xe, ew_ref[e], (((1,), (1,)), ((), ())),
            preferred_element_type=jnp.float32,
        )
    out_ref[...] = acc


def _sc_combine(lg_hbm, cm_hbm, lg_v, cm_v):
    wid = lax.axis_index("s") * NC + lax.axis_index("c")
    base = wid * TPW
    pltpu.sync_copy(lg_hbm.at[:, pl.ds(base, TPW)], lg_v)
    one = jnp.full((L,), 1.0, dtype=jnp.float32)
    zero = jnp.zeros((L,), dtype=jnp.float32)
    kf = jnp.full((L,), float(K), dtype=jnp.float32)
    for i in range(TPW // L):
        ls = [lg_v[e, pl.ds(i * L, L)] for e in range(E)]
        for e in range(E):
            r = zero
            for j in range(E):
                if j == e:
                    continue
                if j < e:
                    beats = ls[j] >= ls[e]
                else:
                    beats = ls[j] > ls[e]
                r = r + jnp.where(beats, one, zero)
            sig = one / (one + jnp.exp(-ls[e]))
            cm_v[e, pl.ds(i * L, L)] = jnp.where(r < kf, sig, zero)
    pltpu.sync_copy(cm_v, cm_hbm.at[:, pl.ds(base, TPW)])


def kernel(inputs, gate_W, gate_b, expert_W, expert_b):
    x = inputs.reshape(N, D)
    gb = gate_b.reshape(E, 1)
    logitsT = pl.pallas_call(
        _gate_kernel,
        grid=(N // BG,),
        in_specs=[
            pl.BlockSpec((BG, D), lambda i: (i, 0)),
            pl.BlockSpec((E, D), lambda i: (0, 0)),
            pl.BlockSpec((E, 1), lambda i: (0, 0)),
        ],
        out_specs=pl.BlockSpec((E, BG), lambda i: (0, i)),
        out_shape=jax.ShapeDtypeStruct((E, N), jnp.float32),
    )(x, gate_W, gb)

    mesh = plsc.VectorSubcoreMesh(core_axis_name="c", subcore_axis_name="s")
    combineT = pl.kernel(
        _sc_combine,
        mesh=mesh,
        out_type=jax.ShapeDtypeStruct((E, N), jnp.float32),
        scratch_types=[
            pltpu.VMEM((E, TPW), jnp.float32),
            pltpu.VMEM((E, TPW), jnp.float32),
        ],
    )(logitsT)

    out = pl.pallas_call(
        _expert_kernel,
        grid=(N // BM,),
        in_specs=[
            pl.BlockSpec((BM, D), lambda i: (i, 0)),
            pl.BlockSpec((E, BM), lambda i: (0, i)),
            pl.BlockSpec((E, D, D), lambda i: (0, 0, 0)),
            pl.BlockSpec((E, D), lambda i: (0, 0)),
        ],
        out_specs=pl.BlockSpec((BM, D), lambda i: (i, 0)),
        out_shape=jax.ShapeDtypeStruct((N, D), jnp.float32),
    )(x, combineT, expert_W, expert_b)
    return out.reshape(B, T, D)


# speedup vs baseline: 1.0934x; 1.0934x over previous
import jax
import jax.numpy as jnp
from jax.experimental import pallas as pl

B, T, D = 4, 2048, 768
E = 8
K = 2
N = B * T
BM = 1024


def _moe_block_kernel(x_ref, gw_ref, gb_ref, ew_ref, eb_ref, out_ref):
    x = x_ref[...]
    logits = jax.lax.dot_general(
        x, gw_ref[...], (((1,), (1,)), ((), ())),
        preferred_element_type=jnp.float32,
    ) + gb_ref[...]
    rank = jnp.zeros((BM, E), dtype=jnp.int32)
    for j in range(E):
        lj = logits[:, j:j + 1]
        gt = (lj > logits[:, :j + 1]).astype(jnp.int32)
        if j + 1 < E:
            ge = (lj >= logits[:, j + 1:]).astype(jnp.int32)
            beats = jnp.concatenate([gt, ge], axis=1)
        else:
            beats = gt
        rank = rank + beats
    combine = jnp.where(rank < K, jax.nn.sigmoid(logits), 0.0)
    acc = jax.lax.dot_general(
        combine, eb_ref[...], (((1,), (0,)), ((), ())),
        preferred_element_type=jnp.float32,
    )
    for e in range(E):
        xe = combine[:, e:e + 1] * x
        acc = acc + jax.lax.dot_general(
            xe, ew_ref[e], (((1,), (1,)), ((), ())),
            preferred_element_type=jnp.float32,
        )
    out_ref[...] = acc


def kernel(inputs, gate_W, gate_b, expert_W, expert_b):
    x = inputs.reshape(N, D)
    gb = gate_b.reshape(1, E)
    out = pl.pallas_call(
        _moe_block_kernel,
        grid=(N // BM,),
        in_specs=[
            pl.BlockSpec((BM, D), lambda i: (i, 0)),
            pl.BlockSpec((E, D), lambda i: (0, 0)),
            pl.BlockSpec((1, E), lambda i: (0, 0)),
            pl.BlockSpec((E, D, D), lambda i: (0, 0, 0)),
            pl.BlockSpec((E, D), lambda i: (0, 0)),
        ],
        out_specs=pl.BlockSpec((BM, D), lambda i: (i, 0)),
        out_shape=jax.ShapeDtypeStruct((N, D), jnp.float32),
    )(x, gate_W, gb, expert_W, expert_b)
    return out.reshape(B, T, D)


# speedup vs baseline: 1.1105x; 1.0156x over previous
import jax
import jax.numpy as jnp
from jax.experimental import pallas as pl

B, T, D = 4, 2048, 768
E = 8
K = 2
N = B * T
BM = 1024


def _moe_block_kernel(x_ref, gw_ref, gb_ref, ew_ref, eb_ref, out_ref):
    x = x_ref[...]
    logits = jax.lax.dot_general(
        x, gw_ref[...], (((1,), (1,)), ((), ())),
        preferred_element_type=jnp.float32,
    ) + gb_ref[...]
    rank = jnp.zeros((BM, E), dtype=jnp.int32)
    col = jax.lax.broadcasted_iota(jnp.int32, (BM, E), 1)
    for j in range(E):
        lj = logits[:, j:j + 1]
        beats = (lj > logits) | ((lj == logits) & (j < col))
        rank = rank + beats.astype(jnp.int32)
    combine = jnp.where(rank < K, jax.nn.sigmoid(logits), 0.0)
    acc = jax.lax.dot_general(
        combine, eb_ref[...], (((1,), (0,)), ((), ())),
        preferred_element_type=jnp.float32,
    )
    for e in range(E):
        xe = combine[:, e:e + 1] * x
        acc = acc + jax.lax.dot_general(
            xe, ew_ref[e], (((1,), (1,)), ((), ())),
            preferred_element_type=jnp.float32,
        )
    out_ref[...] = acc


def kernel(inputs, gate_W, gate_b, expert_W, expert_b):
    x = inputs.reshape(N, D)
    gb = gate_b.reshape(1, E)
    out = pl.pallas_call(
        _moe_block_kernel,
        grid=(N // BM,),
        in_specs=[
            pl.BlockSpec((BM, D), lambda i: (i, 0)),
            pl.BlockSpec((E, D), lambda i: (0, 0)),
            pl.BlockSpec((1, E), lambda i: (0, 0)),
            pl.BlockSpec((E, D, D), lambda i: (0, 0, 0)),
            pl.BlockSpec((E, D), lambda i: (0, 0)),
        ],
        out_specs=pl.BlockSpec((BM, D), lambda i: (i, 0)),
        out_shape=jax.ShapeDtypeStruct((N, D), jnp.float32),
    )(x, gate_W, gb, expert_W, expert_b)
    return out.reshape(B, T, D)
